# Initial kernel scaffold; baseline (speedup 1.0000x reference)
import jax
import jax.numpy as jnp
from jax.experimental import pallas as pl


def kernel(x, Wr, W1, b1, W2, b2):
    raise NotImplementedError("write your pallas kernel here")


# speedup vs baseline: 1.4143x; 1.4143x over previous
import functools

import jax
import jax.numpy as jnp
from jax import lax
from jax.experimental import pallas as pl
from jax.experimental.pallas import tpu as pltpu

NE = 8
TOPK = 2
TM = 256
RT = 256
DEB = 512
LANES = 128


def _router_body(x_ref, wr_ref, e_ref, w_ref):
    xt = x_ref[...]
    logits = jnp.dot(xt, wr_ref[...], preferred_element_type=jnp.float32)
    lane = lax.broadcasted_iota(jnp.int32, logits.shape, 1)
    logits = jnp.where(lane < NE, logits, -1e30)
    m1 = jnp.max(logits, axis=1, keepdims=True)
    a1 = jnp.min(jnp.where(logits >= m1, lane, LANES), axis=1)
    masked = jnp.where(lane == a1[:, None], -1e30, logits)
    m2 = jnp.max(masked, axis=1, keepdims=True)
    a2 = jnp.min(jnp.where(masked >= m2, lane, LANES), axis=1)
    w1 = 1.0 / (1.0 + jnp.exp(m2[:, 0] - m1[:, 0]))
    e_ref[0, :] = a1
    e_ref[1, :] = a2
    w_ref[0, :] = w1
    w_ref[1, :] = 1.0 - w1


def _gelu(h):
    c = 0.7978845608028654
    return 0.5 * h * (1.0 + jnp.tanh(c * (h + 0.044715 * h * h * h)))


def _ffn_body(te_ref, xs_ref, w1_ref, b1_ref, w2_ref, b2_ref, ys_ref):
    k = pl.program_id(1)
    nk = pl.num_programs(1)
    xt = xs_ref[...]
    h = jnp.dot(xt, w1_ref[0], preferred_element_type=jnp.float32) + b1_ref[0]
    h = _gelu(h)

    @pl.when(k == 0)
    def _():
        ys_ref[...] = jnp.zeros_like(ys_ref)

    ys_ref[...] += jnp.dot(h, w2_ref[0], preferred_element_type=jnp.float32)

    @pl.when(k == nk - 1)
    def _():
        ys_ref[...] += b2_ref[0]


def kernel(x, Wr, W1, b1, W2, b2):
    Bb, Tt, D = x.shape
    N = Bb * Tt
    DE = W1.shape[2]
    NT = (TOPK * N + NE * TM) // TM
    NP = NT * TM
    x_flat = x.reshape(N, D)

    wrp = jnp.zeros((D, LANES), jnp.float32).at[:, :NE].set(Wr)
    eidx, wgt = pl.pallas_call(
        _router_body,
        grid=(N // RT,),
        in_specs=[
            pl.BlockSpec((RT, D), lambda i: (i, 0)),
            pl.BlockSpec((D, LANES), lambda i: (0, 0)),
        ],
        out_specs=[
            pl.BlockSpec((TOPK, RT), lambda i: (0, i)),
            pl.BlockSpec((TOPK, RT), lambda i: (0, i)),
        ],
        out_shape=[
            jax.ShapeDtypeStruct((TOPK, N), jnp.int32),
            jax.ShapeDtypeStruct((TOPK, N), jnp.float32),
        ],
    )(x_flat, wrp)

    e_flat = eidx.reshape(-1)
    oh = (e_flat[:, None] == jnp.arange(NE)).astype(jnp.int32)
    csum = jnp.cumsum(oh, axis=0)
    rank = jnp.take_along_axis(csum, e_flat[:, None], axis=1)[:, 0] - 1
    cnt = csum[-1]
    padded = ((cnt + TM - 1) // TM) * TM
    ends = jnp.cumsum(padded)
    starts = ends - padded
    dest = starts[e_flat] + rank
    tok = jnp.concatenate([jnp.arange(N, dtype=jnp.int32)] * TOPK)
    src_idx = jnp.zeros((NP,), jnp.int32).at[dest].set(tok)
    d1, d2 = dest[:N], dest[N:]
    tile_e = jnp.clip(
        jnp.searchsorted(ends, jnp.arange(NT) * TM, side="right"), 0, NE - 1
    ).astype(jnp.int32)

    xs = x_flat[src_idx]

    b1r = b1.reshape(NE, 1, DE)
    b2r = b2.reshape(NE, 1, D)
    ys = pl.pallas_call(
        _ffn_body,
        grid_spec=pltpu.PrefetchScalarGridSpec(
            num_scalar_prefetch=1,
            grid=(NT, DE // DEB),
            in_specs=[
                pl.BlockSpec((TM, D), lambda i, k, te: (i, 0)),
                pl.BlockSpec((1, D, DEB), lambda i, k, te: (te[i], 0, k)),
                pl.BlockSpec((1, 1, DEB), lambda i, k, te: (te[i], 0, k)),
                pl.BlockSpec((1, DEB, D), lambda i, k, te: (te[i], k, 0)),
                pl.BlockSpec((1, 1, D), lambda i, k, te: (te[i], 0, 0)),
            ],
            out_specs=pl.BlockSpec((TM, D), lambda i, k, te: (i, 0)),
        ),
        out_shape=jax.ShapeDtypeStruct((NP, D), jnp.float32),
        compiler_params=pltpu.CompilerParams(
            dimension_semantics=("arbitrary", "arbitrary"),
        ),
    )(tile_e, xs, W1, b1r, W2, b2r)

    out = wgt[0][:, None] * ys[d1] + wgt[1][:, None] * ys[d2]
    return (out.reshape(Bb, Tt, D), jnp.float32(0.0))


# speedup vs baseline: 1.7361x; 1.2275x over previous
import functools

import jax
import jax.numpy as jnp
from jax import lax
from jax.experimental import pallas as pl
from jax.experimental.pallas import tpu as pltpu

NE = 8
TOPK = 2
TM = 256
RT = 256
LANES = 128


def _router_body(x_ref, wr_ref, e_ref, w_ref):
    xt = x_ref[...]
    logits = jnp.dot(xt, wr_ref[...], preferred_element_type=jnp.float32)
    lane = lax.broadcasted_iota(jnp.int32, logits.shape, 1)
    logits = jnp.where(lane < NE, logits, -1e30)
    m1 = jnp.max(logits, axis=1, keepdims=True)
    a1 = jnp.min(jnp.where(logits >= m1, lane, LANES), axis=1)
    masked = jnp.where(lane == a1[:, None], -1e30, logits)
    m2 = jnp.max(masked, axis=1, keepdims=True)
    a2 = jnp.min(jnp.where(masked >= m2, lane, LANES), axis=1)
    w1 = 1.0 / (1.0 + jnp.exp(m2[:, 0] - m1[:, 0]))
    e_ref[0, :] = a1
    e_ref[1, :] = a2
    w_ref[0, :] = w1
    w_ref[1, :] = 1.0 - w1


def _gelu(h):
    c = 0.7978845608028654
    return 0.5 * h * (1.0 + jnp.tanh(c * (h + 0.044715 * h * h * h)))


def _ffn_body(te_ref, xs_ref, w1_ref, b1_ref, w2_ref, b2_ref, ys_ref):
    xt = xs_ref[...]
    h = jnp.dot(xt, w1_ref[0], preferred_element_type=jnp.float32) + b1_ref[0]
    h = _gelu(h)
    ys_ref[...] = jnp.dot(h, w2_ref[0], preferred_element_type=jnp.float32) + b2_ref[0]


def _plan(e_flat, N, NP, NT):
    oh = (e_flat[:, None] == jnp.arange(NE)).astype(jnp.int32)
    csum = jnp.cumsum(oh, axis=0)
    rank = jnp.take_along_axis(csum, e_flat[:, None], axis=1)[:, 0] - 1
    cnt = csum[-1]
    padded = ((cnt + TM - 1) // TM) * TM
    ends = jnp.cumsum(padded)
    starts = ends - padded
    dest = starts[e_flat] + rank
    tok = jnp.concatenate([jnp.arange(N, dtype=jnp.int32)] * TOPK)
    src_idx = jnp.zeros((NP,), jnp.int32).at[dest].set(tok)
    d1, d2 = dest[:N], dest[N:]
    tile_e = jnp.clip(
        jnp.searchsorted(ends, jnp.arange(NT) * TM, side="right"), 0, NE - 1
    ).astype(jnp.int32)
    return src_idx, d1, d2, tile_e


def kernel(x, Wr, W1, b1, W2, b2):
    Bb, Tt, D = x.shape
    N = Bb * Tt
    DE = W1.shape[2]
    NT = (TOPK * N + NE * TM) // TM
    NP = NT * TM
    x_flat = x.reshape(N, D)

    wrp = jnp.zeros((D, LANES), jnp.float32).at[:, :NE].set(Wr)
    eidx, wgt = pl.pallas_call(
        _router_body,
        grid=(N // RT,),
        in_specs=[
            pl.BlockSpec((RT, D), lambda i: (i, 0)),
            pl.BlockSpec((D, LANES), lambda i: (0, 0)),
        ],
        out_specs=[
            pl.BlockSpec((TOPK, RT), lambda i: (0, i)),
            pl.BlockSpec((TOPK, RT), lambda i: (0, i)),
        ],
        out_shape=[
            jax.ShapeDtypeStruct((TOPK, N), jnp.int32),
            jax.ShapeDtypeStruct((TOPK, N), jnp.float32),
        ],
    )(x_flat, wrp)

    src_idx, d1, d2, tile_e = _plan(eidx.reshape(-1), N, NP, NT)

    xs = x_flat[src_idx]

    b1r = b1.reshape(NE, 1, DE)
    b2r = b2.reshape(NE, 1, D)
    ys = pl.pallas_call(
        _ffn_body,
        grid_spec=pltpu.PrefetchScalarGridSpec(
            num_scalar_prefetch=1,
            grid=(NT,),
            in_specs=[
                pl.BlockSpec((TM, D), lambda i, te: (i, 0)),
                pl.BlockSpec((1, D, DE), lambda i, te: (te[i], 0, 0)),
                pl.BlockSpec((1, 1, DE), lambda i, te: (te[i], 0, 0)),
                pl.BlockSpec((1, DE, D), lambda i, te: (te[i], 0, 0)),
                pl.BlockSpec((1, 1, D), lambda i, te: (te[i], 0, 0)),
            ],
            out_specs=pl.BlockSpec((TM, D), lambda i, te: (i, 0)),
        ),
        out_shape=jax.ShapeDtypeStruct((NP, D), jnp.float32),
        compiler_params=pltpu.CompilerParams(
            dimension_semantics=("arbitrary",),
        ),
    )(tile_e, xs, W1, b1r, W2, b2r)

    out = wgt[0][:, None] * ys[d1] + wgt[1][:, None] * ys[d2]
    return (out.reshape(Bb, Tt, D), jnp.float32(0.0))
